# baseline (speedup 1.0000x reference)
import jax
import jax.numpy as jnp
from jax.experimental import pallas as pl
from jax.experimental.pallas import tpu as pltpu


def _fused_ct_kernel(f_ref, v_ref,
                     w_e_ref, b_e_ref,
                     w_db_ref, b_db_ref,
                     w_ch_ref, b_ch_ref,
                     w_a1_ref, b_a1_ref, w_a2_ref, b_a2_ref,
                     w_a3_ref, b_a3_ref,
                     o_atn_ref, o_filt_ref, o_new_ref):
    f32, bf16 = jnp.float32, jnp.bfloat16

    def leaky(x):
        return jnp.where(x > 0, x, 0.2 * x)

    def im2col3(x):
        c, t = x.shape
        z = jnp.zeros((c, 1), x.dtype)
        prev = jnp.concatenate([z, x[:, :t - 1]], axis=1)
        nxt = jnp.concatenate([x[:, 1:], z], axis=1)
        return jnp.concatenate([prev, x, nxt], axis=0).astype(bf16)

    def conv3(cols_bf16, w_ref, b_ref):
        y = jnp.dot(w_ref[...], cols_bf16, preferred_element_type=f32)
        return y + b_ref[...]

    f = f_ref[...]
    v = v_ref[...]
    C = o_new_ref.shape[0]

    fusion = leaky(conv3(im2col3(f), w_e_ref, b_e_ref))
    db = leaky(conv3(im2col3(fusion), w_db_ref, b_db_ref))
    new_feat = db[:C, :]
    bwa = db[C:, :]

    chfeat = jnp.mean(v, axis=1, keepdims=True)
    ca = leaky(jnp.dot(w_ch_ref[...], chfeat.astype(bf16),
                       preferred_element_type=f32) + b_ch_ref[...])

    ca_n = ca * jax.lax.rsqrt(jnp.sum(ca * ca, axis=0, keepdims=True))
    bwa_n = bwa * jax.lax.rsqrt(jnp.sum(bwa * bwa, axis=0, keepdims=True))
    temp = jnp.sum(ca_n * bwa_n, axis=0, keepdims=True)
    filt = jax.nn.sigmoid(bwa * temp) * v

    a1 = leaky(conv3(im2col3(filt), w_a1_ref, b_a1_ref))
    a2 = leaky(conv3(im2col3(a1), w_a2_ref, b_a2_ref))
    scores = jnp.dot(w_a3_ref[...], a2,
                     preferred_element_type=f32) + b_a3_ref[...]

    o_atn_ref[...] = jax.nn.sigmoid(scores).astype(o_atn_ref.dtype)
    o_filt_ref[...] = filt.astype(o_filt_ref.dtype)
    o_new_ref[...] = new_feat.astype(o_new_ref.dtype)


def kernel(ae_e_w, ae_e_b, ae_d_w, ae_d_b, bit_wise_w, bit_wise_b,
           channel_w, channel_b, attn1_w, attn1_b, attn2_w, attn2_b,
           attn3_w, attn3_b, vfeat, ffeat):
    B, C, T = vfeat.shape
    E = channel_w.shape[2]
    H = attn1_w.shape[2]
    C2 = C // 2
    bf16 = jnp.bfloat16

    flat = lambda w: jnp.transpose(w, (2, 0, 1)).reshape(w.shape[2], -1).astype(bf16)
    colb = lambda b: b.reshape(-1, 1)
    w_e, b_e = flat(ae_e_w), colb(ae_e_b)
    w_db = jnp.concatenate([flat(ae_d_w), flat(bit_wise_w)], axis=0)
    b_db = jnp.concatenate([colb(ae_d_b), colb(bit_wise_b)], axis=0)
    w_ch = jnp.transpose(channel_w[1]).astype(bf16)
    b_ch = colb(channel_b)
    w_a1, b_a1 = flat(attn1_w), colb(attn1_b)
    w_a2, b_a2 = flat(attn2_w), colb(attn2_b)
    w_a3 = jnp.transpose(attn3_w[0])
    b_a3 = attn3_b

    resident = lambda shape: pl.BlockSpec(shape, lambda b: (0, 0))
    per_batch = lambda shape: pl.BlockSpec((None,) + shape, lambda b: (b, 0, 0))

    x_atn, filt, new_feat = pl.pallas_call(
        _fused_ct_kernel,
        out_shape=(
            jax.ShapeDtypeStruct((B, 1, T), vfeat.dtype),
            jax.ShapeDtypeStruct((B, E, T), vfeat.dtype),
            jax.ShapeDtypeStruct((B, C, T), vfeat.dtype),
        ),
        grid=(B,),
        in_specs=[
            per_batch((C, T)),
            per_batch((C, T)),
            resident((C2, 3 * C)), resident((C2, 1)),
            resident((C + E, 3 * C2)), resident((C + E, 1)),
            resident((E, C)), resident((E, 1)),
            resident((H, 3 * E)), resident((H, 1)),
            resident((H, 3 * H)), resident((H, 1)),
            resident((1, H)), resident((1, 1)),
        ],
        out_specs=[
            per_batch((1, T)),
            per_batch((E, T)),
            per_batch((C, T)),
        ],
        compiler_params=pltpu.CompilerParams(
            dimension_semantics=("parallel",),
            vmem_limit_bytes=100 * 1024 * 1024,
        ),
    )(ffeat, vfeat, w_e, b_e, w_db, b_db, w_ch, b_ch,
      w_a1, b_a1, w_a2, b_a2, w_a3, b_a3)

    return x_atn, filt, new_feat, vfeat
